# baseline (speedup 1.0000x reference)
import functools

import jax
import jax.numpy as jnp
from jax import lax
from jax.experimental import pallas as pl
from jax.experimental.pallas import tpu as pltpu
from jax.experimental.pallas import tpu_sc as plsc

NC = 2
NS = 16
NW = NC * NS
LANES = 16
CHUNK = 128
TCB = 131072
SPLIT = 5 * TCB
PER_TILE = 85
Y2PAD = NW * PER_TILE * CHUNK


@functools.lru_cache(maxsize=None)
def _build_matvec(vocab: int, dim: int):
    assert SPLIT % TCB == 0

    def body(t_ref, w_ref, b_ref, y_ref):
        t = t_ref[...]
        w = w_ref[...]
        y_ref[...] = jnp.sum(t * w, axis=0) + b_ref[0]

    return pl.pallas_call(
        body,
        grid=(SPLIT // TCB,),
        in_specs=[
            pl.BlockSpec((dim, TCB), lambda i: (0, i)),
            pl.BlockSpec((dim, 1), lambda i: (0, 0)),
            pl.BlockSpec(memory_space=pltpu.SMEM),
        ],
        out_specs=pl.BlockSpec((TCB,), lambda i: (i,)),
        out_shape=jax.ShapeDtypeStruct((SPLIT,), jnp.float32),
        compiler_params=pltpu.CompilerParams(
            dimension_semantics=("arbitrary",)),
    )


@functools.lru_cache(maxsize=None)
def _build_sc_matvec(vocab: int, dim: int):
    padded_w = ((vocab + CHUNK - 1) // CHUNK) * CHUNK
    last_c0 = padded_w - CHUNK
    slab = PER_TILE * CHUNK
    mesh = plsc.VectorSubcoreMesh(core_axis_name="c", subcore_axis_name="s")

    @functools.partial(
        pl.kernel,
        out_type=jax.ShapeDtypeStruct((Y2PAD,), jnp.float32),
        mesh=mesh,
        scratch_types=[
            pltpu.VMEM((2, 2, 8, CHUNK), jnp.float32),
            pltpu.VMEM((slab,), jnp.float32),
            pltpu.VMEM((LANES,), jnp.float32),
            pltpu.VMEM((LANES,), jnp.float32),
            pltpu.SemaphoreType.DMA,
            pltpu.SemaphoreType.DMA,
        ],
        compiler_params=pltpu.CompilerParams(
            needs_layout_passes=False, use_tc_tiling_on_sc=True),
    )
    def sc_mv(tab_hbm, w_hbm, b_hbm, y2_hbm, buf, y2loc, w_v, b_v, s0, s1):
        wid = lax.axis_index("s") * NC + lax.axis_index("c")
        base_k = wid * PER_TILE
        sems = (s0, s1)

        pltpu.sync_copy(w_hbm, w_v)
        pltpu.sync_copy(b_hbm, b_v)
        w = w_v[...]
        bias = b_v[...]

        def c0_of(k):
            return jnp.minimum(SPLIT + (base_k + k) * CHUNK, last_c0)

        def fire(k, slot):
            c0 = c0_of(k)
            pltpu.async_copy(tab_hbm.at[pl.ds(0, 8), pl.ds(c0, CHUNK)],
                             buf.at[slot, 0], sems[slot])
            pltpu.async_copy(tab_hbm.at[pl.ds(8, 8), pl.ds(c0, CHUNK)],
                             buf.at[slot, 1], sems[slot])

        def drain(slot):
            for _ in range(2):
                pltpu.make_async_copy(
                    tab_hbm.at[pl.ds(0, 8), pl.ds(0, CHUNK)],
                    buf.at[slot, 0], sems[slot]).wait()

        fire(0, 0)

        def pair(p, carry):
            for slot in (0, 1):
                k = p * 2 + slot

                @pl.when(k < PER_TILE)
                def _process():
                    drain(slot)

                    @pl.when(k + 1 < PER_TILE)
                    def _prefetch():
                        fire(k + 1, 1 - slot)

                    for g in range(CHUNK // LANES):
                        acc = bias
                        for j in range(dim):
                            col = buf[slot, j // 8, j % 8,
                                      pl.ds(g * LANES, LANES)]
                            acc = acc + col * w[j]
                        y2loc[pl.ds(k * CHUNK + g * LANES, LANES)] = acc

            return carry

        lax.fori_loop(0, (PER_TILE + 1) // 2, pair, 0)

        pltpu.sync_copy(y2loc, y2_hbm.at[pl.ds(wid * slab, slab)])

    return sc_mv


@functools.lru_cache(maxsize=None)
def _build_gather(vocab: int, batch: int):
    b_per_w = batch // NW
    n_chunks = b_per_w // CHUNK
    assert b_per_w % CHUNK == 0

    mesh = plsc.VectorSubcoreMesh(core_axis_name="c", subcore_axis_name="s")

    @functools.partial(
        pl.kernel,
        out_type=jax.ShapeDtypeStruct((batch,), jnp.float32),
        mesh=mesh,
        scratch_types=[
            pltpu.VMEM((n_chunks, CHUNK), jnp.int32),
            pltpu.VMEM((n_chunks, CHUNK), jnp.int32),
            pltpu.VMEM((n_chunks, CHUNK), jnp.int32),
            pltpu.VMEM((b_per_w,), jnp.float32),
            pltpu.VMEM((b_per_w,), jnp.float32),
            pltpu.VMEM((b_per_w,), jnp.float32),
            pltpu.SemaphoreType.DMA,
        ],
        compiler_params=pltpu.CompilerParams(
            needs_layout_passes=False, use_tc_tiling_on_sc=False),
    )
    def sc_gather(idx_hbm, y1_hbm, y2_hbm, out_hbm,
                  idx_v, idx1_v, idx2_v, g1_v, g2_v, out_v, sem):
        wid = lax.axis_index("s") * NC + lax.axis_index("c")
        base = wid * b_per_w

        pltpu.sync_copy(idx_hbm.at[wid], idx_v)

        for k in range(n_chunks):
            for g in range(CHUNK // LANES):
                v = idx_v[k, pl.ds(g * LANES, LANES)]
                idx1_v[k, pl.ds(g * LANES, LANES)] = jnp.minimum(v, SPLIT - 1)
                idx2_v[k, pl.ds(g * LANES, LANES)] = jnp.maximum(v - SPLIT, 0)

        copies = [
            pltpu.async_copy(y1_hbm.at[idx1_v.at[k]],
                             g1_v.at[pl.ds(k * CHUNK, CHUNK)], sem)
            for k in range(n_chunks)
        ] + [
            pltpu.async_copy(y2_hbm.at[idx2_v.at[k]],
                             g2_v.at[pl.ds(k * CHUNK, CHUNK)], sem)
            for k in range(n_chunks)
        ]
        for c in copies:
            c.wait()

        for k in range(n_chunks):
            for g in range(CHUNK // LANES):
                sl = pl.ds(k * CHUNK + g * LANES, LANES)
                v = idx_v[k, pl.ds(g * LANES, LANES)]
                out_v[sl] = jnp.where(v < SPLIT, g1_v[sl], g2_v[sl])

        pltpu.sync_copy(out_v, out_hbm.at[pl.ds(base, b_per_w)])

    return sc_gather


def kernel(x, table, fc_w, fc_b):
    batch = x.shape[0]
    vocab, dim = table.shape
    table_t = table.T
    w_col = fc_w.reshape(dim, 1).astype(jnp.float32)
    w_vec = fc_w.reshape(dim).astype(jnp.float32)
    b_vec = jnp.broadcast_to(fc_b.astype(jnp.float32), (LANES,))
    y1 = _build_matvec(vocab, dim)(table_t, w_col, fc_b.astype(jnp.float32))
    y2 = _build_sc_matvec(vocab, dim)(table_t, w_vec, b_vec)
    idx = x.astype(jnp.int32).reshape(NW, batch // NW // CHUNK, CHUNK)
    out = _build_gather(vocab, batch)(idx, y1, y2)
    return out.reshape(batch, 1)


# speedup vs baseline: 3.0421x; 3.0421x over previous
import functools

import jax
import jax.numpy as jnp
from jax import lax
from jax.experimental import pallas as pl
from jax.experimental.pallas import tpu as pltpu
from jax.experimental.pallas import tpu_sc as plsc

NC = 2
NS = 16
NW = NC * NS
CHUNK = 128
COLS = 262144


@functools.lru_cache(maxsize=None)
def _build_matvec(vocab: int, dim: int):
    grid = (vocab + COLS - 1) // COLS

    def body(t_ref, w_ref, b_ref, y_ref):
        t = t_ref[...]
        w = w_ref[...]
        y_ref[...] = jnp.sum(t * w, axis=0) + b_ref[0]

    return pl.pallas_call(
        body,
        grid=(grid,),
        in_specs=[
            pl.BlockSpec((dim, COLS), lambda i: (0, i)),
            pl.BlockSpec((dim, 1), lambda i: (0, 0)),
            pl.BlockSpec(memory_space=pltpu.SMEM),
        ],
        out_specs=pl.BlockSpec((COLS,), lambda i: (i,)),
        out_shape=jax.ShapeDtypeStruct((9 * 1024 * 1024, ), jnp.float32),
        compiler_params=pltpu.CompilerParams(
            dimension_semantics=("arbitrary",)),
    )


@functools.lru_cache(maxsize=None)
def _build_gather(vocab: int, batch: int):
    b_per_w = batch // NW
    n_chunks = b_per_w // CHUNK
    assert b_per_w % CHUNK == 0

    mesh = plsc.VectorSubcoreMesh(core_axis_name="c", subcore_axis_name="s")

    @functools.partial(
        pl.kernel,
        out_type=jax.ShapeDtypeStruct((batch,), jnp.float32),
        mesh=mesh,
        scratch_types=[
            pltpu.VMEM((n_chunks, CHUNK), jnp.int32),
            pltpu.VMEM((b_per_w,), jnp.float32),
            pltpu.SemaphoreType.DMA,
        ],
        compiler_params=pltpu.CompilerParams(
            needs_layout_passes=False, use_tc_tiling_on_sc=False),
    )
    def sc_kernel(idx_hbm, y_hbm, out_hbm, idx_v, out_v, sem):
        wid = lax.axis_index("s") * NC + lax.axis_index("c")
        base = wid * b_per_w

        pltpu.sync_copy(idx_hbm.at[wid], idx_v)

        copies = [
            pltpu.async_copy(
                y_hbm.at[idx_v.at[k]],
                out_v.at[pl.ds(k * CHUNK, CHUNK)],
                sem,
            )
            for k in range(n_chunks)
        ]
        for c in copies:
            c.wait()

        pltpu.sync_copy(out_v, out_hbm.at[pl.ds(base, b_per_w)])

    return sc_kernel


def kernel(x, table, fc_w, fc_b):
    batch = x.shape[0]
    vocab, dim = table.shape
    table_t = table.T
    w = fc_w.reshape(dim, 1).astype(jnp.float32)
    y = _build_matvec(vocab, dim)(table_t, w, fc_b.astype(jnp.float32))
    idx = x.astype(jnp.int32).reshape(NW, batch // NW // CHUNK, CHUNK)
    out = _build_gather(vocab, batch)(idx, y)
    return out.reshape(batch, 1)
